# baseline (speedup 1.0000x reference)
import functools

import jax
import jax.numpy as jnp
from jax import lax
from jax.experimental import pallas as pl
from jax.experimental.pallas import tpu as pltpu
from jax.experimental.pallas import tpu_sc as plsc

N = 10000
F_IN = 128
HID = 64
NCLASS = 40
NCP = 48
E = 320000
NC, NS = 2, 16
NW = NC * NS
K = 128
NCH = 80
EPAD = NW * NCH * K
NPAD = 10240
RPT = NPAD // NS

f32 = jnp.float32


def _mesh():
    return plsc.VectorSubcoreMesh(
        core_axis_name="c", subcore_axis_name="s", num_cores=NC, num_subcores=NS
    )


def _deg_body(dst_ref, out_ref, idx_v, ones_v, zbuf, acc, ss0, ss1):
    c = lax.axis_index("c")
    s = lax.axis_index("s")
    wid = c * NS + s
    one16 = jnp.ones((16,), f32)
    zero16 = jnp.zeros((16,), f32)

    def fill_ones(i, _):
        ones_v[i, :] = one16
        return 0

    lax.fori_loop(0, K, fill_ones, 0)

    def fill_zero(i, _):
        zbuf[i, :] = zero16
        return 0

    lax.fori_loop(0, RPT, fill_zero, 0)
    pltpu.sync_copy(zbuf, acc.at[pl.ds(s * RPT, RPT)])
    pltpu.sync_copy(dst_ref.at[wid], idx_v)
    plsc.subcore_barrier()

    ssem = (ss0, ss1)

    def st(j, b):
        pltpu.async_copy(ones_v, acc.at[idx_v.at[j]], ssem[b], add=True)

    def ws(b):
        pltpu.make_async_copy(ones_v, acc.at[idx_v.at[0]], ssem[b]).wait()

    st(0, 0)

    def chunk(j2, _):
        j = 2 * j2 + 1
        st(j, 1)
        ws(0)
        st(j + 1, 0)
        ws(1)
        return 0

    lax.fori_loop(0, (NCH - 2) // 2, chunk, 0)
    st(NCH - 1, 1)
    ws(0)
    ws(1)
    plsc.subcore_barrier()
    pltpu.sync_copy(acc.at[pl.ds(s * RPT, RPT)], out_ref.at[c, pl.ds(s * RPT, RPT)])


def _deg_call(dst3):
    return pl.kernel(
        _deg_body,
        out_type=jax.ShapeDtypeStruct((NC, NPAD, 16), f32),
        mesh=_mesh(),
        compiler_params=pltpu.CompilerParams(use_tc_tiling_on_sc=False),
        scratch_types=[
            pltpu.VMEM((NCH, K), jnp.int32),
            pltpu.VMEM((K, 16), f32),
            pltpu.VMEM((RPT, 16), f32),
            pltpu.VMEM_SHARED((NPAD, 16), f32),
            pltpu.SemaphoreType.DMA,
            pltpu.SemaphoreType.DMA,
        ],
    )(dst3)


def _make_agg_body(D):
    def body(
        tbl_ref, src_ref, dst_ref, out_ref,
        src_v, dst_v, buf0, buf1, buf2, buf3, zbuf, acc,
        gs0, gs1, gs2, gs3, ss0, ss1, ss2, ss3,
    ):
        c = lax.axis_index("c")
        s = lax.axis_index("s")
        wid = c * NS + s
        bufs = (buf0, buf1, buf2, buf3)
        gsem = (gs0, gs1, gs2, gs3)
        ssem = (ss0, ss1, ss2, ss3)
        zero16 = jnp.zeros((16,), f32)

        def fill_zero(i, _):
            for l in range(D // 16):
                zbuf[i, pl.ds(l * 16, 16)] = zero16
            return 0

        lax.fori_loop(0, K, fill_zero, 0)
        for t in range(RPT // K):
            pltpu.sync_copy(zbuf, acc.at[pl.ds(s * RPT + t * K, K)])
        pltpu.sync_copy(src_ref.at[wid], src_v)
        pltpu.sync_copy(dst_ref.at[wid], dst_v)
        plsc.subcore_barrier()

        def sg(j, b):
            pltpu.async_copy(tbl_ref.at[src_v.at[j]], bufs[b], gsem[b])

        def wg(b):
            pltpu.make_async_copy(tbl_ref.at[src_v.at[0]], bufs[b], gsem[b]).wait()

        def st(j, b):
            pltpu.async_copy(bufs[b], acc.at[dst_v.at[j]], ssem[b], add=True)

        def ws(b):
            pltpu.make_async_copy(bufs[b], acc.at[dst_v.at[0]], ssem[b]).wait()

        sg(0, 0)
        sg(1, 1)
        wg(0)
        st(0, 0)
        sg(2, 2)
        wg(1)
        st(1, 1)
        sg(3, 3)

        def step(j, b):
            wg(b)
            st(j, b)
            bn = (b + 2) % 4
            ws(bn)
            sg(j + 2, bn)

        def main4(j4, _):
            base = 2 + 4 * j4
            step(base + 0, 2)
            step(base + 1, 3)
            step(base + 2, 0)
            step(base + 3, 1)
            return 0

        lax.fori_loop(0, (NCH - 4) // 4, main4, 0)
        wg(2)
        st(NCH - 2, 2)
        wg(3)
        st(NCH - 1, 3)
        ws(0)
        ws(1)
        ws(2)
        ws(3)
        plsc.subcore_barrier()
        pltpu.sync_copy(
            acc.at[pl.ds(s * RPT, RPT)], out_ref.at[c, pl.ds(s * RPT, RPT)]
        )

    return body


def _agg_call(table, src3, dst3, D):
    return pl.kernel(
        _make_agg_body(D),
        out_type=jax.ShapeDtypeStruct((NC, NPAD, D), f32),
        mesh=_mesh(),
        compiler_params=pltpu.CompilerParams(use_tc_tiling_on_sc=False),
        scratch_types=[
            pltpu.VMEM((NCH, K), jnp.int32),
            pltpu.VMEM((NCH, K), jnp.int32),
            pltpu.VMEM((K, D), f32),
            pltpu.VMEM((K, D), f32),
            pltpu.VMEM((K, D), f32),
            pltpu.VMEM((K, D), f32),
            pltpu.VMEM((K, D), f32),
            pltpu.VMEM_SHARED((NPAD, D), f32),
            pltpu.SemaphoreType.DMA,
            pltpu.SemaphoreType.DMA,
            pltpu.SemaphoreType.DMA,
            pltpu.SemaphoreType.DMA,
            pltpu.SemaphoreType.DMA,
            pltpu.SemaphoreType.DMA,
            pltpu.SemaphoreType.DMA,
            pltpu.SemaphoreType.DMA,
        ],
    )(table, src3, dst3)


def _tc1(x, W1, degparts):
    def body(x_ref, w_ref, dp_ref, h_ref, dinv_ref):
        dp = dp_ref[...]
        deg = dp[0, :N, 0:1] + dp[1, :N, 0:1] + 1.0
        dinv = lax.rsqrt(deg)
        h = jnp.dot(x_ref[...], w_ref[...], preferred_element_type=f32)
        h_ref[...] = h * dinv
        dinv_ref[...] = dinv

    return pl.pallas_call(
        body,
        out_shape=(
            jax.ShapeDtypeStruct((N, HID), f32),
            jax.ShapeDtypeStruct((N, 1), f32),
        ),
    )(x, W1, degparts)


def _tc2(h1s, agg, dinv, b1, W2p):
    def body(h_ref, a_ref, d_ref, b_ref, w_ref, o_ref):
        a = a_ref[...]
        dinv = d_ref[...]
        out1 = jnp.maximum(dinv * (a[0, :N] + a[1, :N] + h_ref[...]) + b_ref[...], 0.0)
        h2 = jnp.dot(out1, w_ref[...], preferred_element_type=f32)
        o_ref[...] = h2 * dinv

    return pl.pallas_call(body, out_shape=jax.ShapeDtypeStruct((N, NCP), f32))(
        h1s, agg, dinv, b1, W2p
    )


def _tc3(h2s, agg, dinv, b2):
    def body(h_ref, a_ref, d_ref, b_ref, o_ref):
        a = a_ref[...]
        logits = d_ref[...] * (a[0, :N] + a[1, :N] + h_ref[...])
        l = logits[:, :NCLASS] + b_ref[...]
        m = jnp.max(l, axis=-1, keepdims=True)
        e = jnp.exp(l - m)
        o_ref[...] = e / jnp.sum(e, axis=-1, keepdims=True)

    return pl.pallas_call(body, out_shape=jax.ShapeDtypeStruct((N, NCLASS), f32))(
        h2s, agg, dinv, b2
    )


@jax.jit
def kernel(x, edge_index, W1, b1, W2, b2):
    pad = EPAD - E
    src3 = jnp.concatenate(
        [edge_index[0], jnp.zeros((pad,), jnp.int32)]
    ).reshape(NW, NCH, K)
    dst3 = jnp.concatenate(
        [edge_index[1], jnp.full((pad,), NPAD - 1, jnp.int32)]
    ).reshape(NW, NCH, K)
    degparts = _deg_call(dst3)
    h1s, dinv = _tc1(x, W1, degparts)
    agg1 = _agg_call(h1s, src3, dst3, HID)
    W2p = jnp.pad(W2, ((0, 0), (0, NCP - NCLASS)))
    h2s = _tc2(h1s, agg1, dinv, b1.reshape(1, HID), W2p)
    agg2 = _agg_call(h2s, src3, dst3, NCP)
    return _tc3(h2s, agg2, dinv, b2.reshape(1, NCLASS))


# speedup vs baseline: 1.0101x; 1.0101x over previous
import functools

import jax
import jax.numpy as jnp
from jax import lax
from jax.experimental import pallas as pl
from jax.experimental.pallas import tpu as pltpu
from jax.experimental.pallas import tpu_sc as plsc

N = 10000
F_IN = 128
HID = 64
NCLASS = 40
NCP = 48
E = 320000
NC, NS = 2, 16
NW = NC * NS
K = 128
NCH = 80
EPAD = NW * NCH * K
NPAD = 10240
RPT = NPAD // NS

f32 = jnp.float32


def _mesh():
    return plsc.VectorSubcoreMesh(
        core_axis_name="c", subcore_axis_name="s", num_cores=NC, num_subcores=NS
    )


def _deg_body(dst_ref, out_ref, idx_v, ones_v, zbuf, acc, ss0, ss1):
    c = lax.axis_index("c")
    s = lax.axis_index("s")
    wid = c * NS + s
    one16 = jnp.ones((16,), f32)
    zero16 = jnp.zeros((16,), f32)

    def fill_ones(i, _):
        ones_v[i, :] = one16
        return 0

    lax.fori_loop(0, K, fill_ones, 0)

    def fill_zero(i, _):
        zbuf[i, :] = zero16
        return 0

    lax.fori_loop(0, RPT, fill_zero, 0)
    pltpu.sync_copy(zbuf, acc.at[pl.ds(s * RPT, RPT)])
    pltpu.sync_copy(dst_ref.at[wid], idx_v)
    plsc.subcore_barrier()

    ssem = (ss0, ss1)

    def st(j, b):
        pltpu.async_copy(ones_v, acc.at[idx_v.at[j]], ssem[b], add=True)

    def ws(b):
        pltpu.make_async_copy(ones_v, acc.at[idx_v.at[0]], ssem[b]).wait()

    st(0, 0)

    def chunk(j2, _):
        j = 2 * j2 + 1
        st(j, 1)
        ws(0)
        st(j + 1, 0)
        ws(1)
        return 0

    lax.fori_loop(0, (NCH - 2) // 2, chunk, 0)
    st(NCH - 1, 1)
    ws(0)
    ws(1)
    plsc.subcore_barrier()
    pltpu.sync_copy(acc.at[pl.ds(s * RPT, RPT)], out_ref.at[c, pl.ds(s * RPT, RPT)])


def _deg_call(dst3):
    return pl.kernel(
        _deg_body,
        out_type=jax.ShapeDtypeStruct((NC, NPAD, 16), f32),
        mesh=_mesh(),
        compiler_params=pltpu.CompilerParams(use_tc_tiling_on_sc=False),
        scratch_types=[
            pltpu.VMEM((NCH, K), jnp.int32),
            pltpu.VMEM((K, 16), f32),
            pltpu.VMEM((RPT, 16), f32),
            pltpu.VMEM_SHARED((NPAD, 16), f32),
            pltpu.SemaphoreType.DMA,
            pltpu.SemaphoreType.DMA,
        ],
    )(dst3)


def _make_agg_body(D):
    def body(
        tbl_ref, src_ref, dst_ref, out_ref,
        src_v, dst_v, buf0, buf1, buf2, buf3, zbuf, acc,
        gs0, gs1, gs2, gs3, ss0, ss1, ss2, ss3,
    ):
        c = lax.axis_index("c")
        s = lax.axis_index("s")
        wid = c * NS + s
        bufs = (buf0, buf1, buf2, buf3)
        gsem = (gs0, gs1, gs2, gs3)
        ssem = (ss0, ss1, ss2, ss3)
        zero16 = jnp.zeros((16,), f32)

        def fill_zero(i, _):
            for l in range(D // 16):
                zbuf[i, pl.ds(l * 16, 16)] = zero16
            return 0

        lax.fori_loop(0, K, fill_zero, 0)
        for t in range(RPT // K):
            pltpu.sync_copy(zbuf, acc.at[pl.ds(s * RPT + t * K, K)])
        pltpu.sync_copy(src_ref.at[wid], src_v)
        pltpu.sync_copy(dst_ref.at[wid], dst_v)
        plsc.subcore_barrier()

        def sg(j, b):
            pltpu.async_copy(tbl_ref.at[src_v.at[j]], bufs[b], gsem[b])

        def wg(b):
            pltpu.make_async_copy(tbl_ref.at[src_v.at[0]], bufs[b], gsem[b]).wait()

        def st(j, b):
            pltpu.async_copy(bufs[b], acc.at[dst_v.at[j]], ssem[b], add=True)

        def ws(b):
            pltpu.make_async_copy(bufs[b], acc.at[dst_v.at[0]], ssem[b]).wait()

        sg(0, 0)
        sg(1, 1)
        wg(0)
        st(0, 0)
        sg(2, 2)
        wg(1)
        st(1, 1)
        sg(3, 3)

        def step(j, b):
            wg(b)
            st(j, b)
            bn = (b + 2) % 4
            ws(bn)
            sg(j + 2, bn)

        def main4(j4, _):
            base = 2 + 4 * j4
            step(base + 0, 2)
            step(base + 1, 3)
            step(base + 2, 0)
            step(base + 3, 1)
            return 0

        lax.fori_loop(0, (NCH - 4) // 4, main4, 0)
        wg(2)
        st(NCH - 2, 2)
        wg(3)
        st(NCH - 1, 3)
        ws(0)
        ws(1)
        ws(2)
        ws(3)
        plsc.subcore_barrier()
        pltpu.sync_copy(
            acc.at[pl.ds(s * RPT, RPT)], out_ref.at[c, pl.ds(s * RPT, RPT)]
        )

    return body


def _agg_call(table, src3, dst3, D):
    return pl.kernel(
        _make_agg_body(D),
        out_type=jax.ShapeDtypeStruct((NC, NPAD, D), f32),
        mesh=_mesh(),
        compiler_params=pltpu.CompilerParams(use_tc_tiling_on_sc=False),
        scratch_types=[
            pltpu.VMEM((NCH, K), jnp.int32),
            pltpu.VMEM((NCH, K), jnp.int32),
            pltpu.VMEM((K, D), f32),
            pltpu.VMEM((K, D), f32),
            pltpu.VMEM((K, D), f32),
            pltpu.VMEM((K, D), f32),
            pltpu.VMEM((K, D), f32),
            pltpu.VMEM_SHARED((NPAD, D), f32),
            pltpu.SemaphoreType.DMA,
            pltpu.SemaphoreType.DMA,
            pltpu.SemaphoreType.DMA,
            pltpu.SemaphoreType.DMA,
            pltpu.SemaphoreType.DMA,
            pltpu.SemaphoreType.DMA,
            pltpu.SemaphoreType.DMA,
            pltpu.SemaphoreType.DMA,
        ],
    )(table, src3, dst3)


def _tc1(x, W1, degparts):
    def body(x_ref, w_ref, dp_ref, h_ref, dinv_ref):
        dp = dp_ref[...]
        deg = dp[0, :N, 0:1] + dp[1, :N, 0:1] + 1.0
        dinv = lax.rsqrt(deg)
        h = jnp.dot(x_ref[...], w_ref[...], preferred_element_type=f32)
        h_ref[...] = h * dinv
        dinv_ref[...] = dinv

    return pl.pallas_call(
        body,
        out_shape=(
            jax.ShapeDtypeStruct((N, HID), f32),
            jax.ShapeDtypeStruct((N, 1), f32),
        ),
    )(x, W1, degparts)


def _tc2(h1s, agg, dinv, b1, W2p):
    def body(h_ref, a_ref, d_ref, b_ref, w_ref, o_ref):
        a = a_ref[...]
        dinv = d_ref[...]
        out1 = jnp.maximum(dinv * (a[0, :N] + a[1, :N] + h_ref[...]) + b_ref[...], 0.0)
        h2 = jnp.dot(out1, w_ref[...], preferred_element_type=f32)
        o_ref[...] = h2 * dinv

    return pl.pallas_call(body, out_shape=jax.ShapeDtypeStruct((N, NCP), f32))(
        h1s, agg, dinv, b1, W2p
    )


def _tc3(h2s, agg, dinv, b2):
    def body(h_ref, a_ref, d_ref, b_ref, o_ref):
        a = a_ref[...]
        logits = d_ref[...] * (a[0, :N] + a[1, :N] + h_ref[...])
        l = logits[:, :NCLASS] + b_ref[...]
        m = jnp.max(l, axis=-1, keepdims=True)
        e = jnp.exp(l - m)
        o_ref[...] = e / jnp.sum(e, axis=-1, keepdims=True)

    return pl.pallas_call(body, out_shape=jax.ShapeDtypeStruct((N, NCLASS), f32))(
        h2s, agg, dinv, b2
    )


@jax.jit
def kernel(x, edge_index, W1, b1, W2, b2):
    pad = EPAD - E
    src3 = jnp.concatenate(
        [edge_index[0], jnp.zeros((pad,), jnp.int32)]
    ).reshape(NW, NCH, K)
    dst3 = jnp.concatenate(
        [edge_index[1], N + jnp.arange(pad, dtype=jnp.int32) % (NPAD - N)]
    ).reshape(NW, NCH, K)
    degparts = _deg_call(dst3)
    h1s, dinv = _tc1(x, W1, degparts)
    agg1 = _agg_call(h1s, src3, dst3, HID)
    W2p = jnp.pad(W2, ((0, 0), (0, NCP - NCLASS)))
    h2s = _tc2(h1s, agg1, dinv, b1.reshape(1, HID), W2p)
    agg2 = _agg_call(h2s, src3, dst3, NCP)
    return _tc3(h2s, agg2, dinv, b2.reshape(1, NCLASS))


# speedup vs baseline: 1.0596x; 1.0490x over previous
import functools

import jax
import jax.numpy as jnp
from jax import lax
from jax.experimental import pallas as pl
from jax.experimental.pallas import tpu as pltpu
from jax.experimental.pallas import tpu_sc as plsc

N = 10000
F_IN = 128
HID = 64
NCLASS = 40
NCP = 48
E = 320000
NC, NS = 2, 16
NW = NC * NS
K = 128
NCH = 80
EPAD = NW * NCH * K
NPAD = 10240
RPT = NPAD // NS

f32 = jnp.float32


def _mesh():
    return plsc.VectorSubcoreMesh(
        core_axis_name="c", subcore_axis_name="s", num_cores=NC, num_subcores=NS
    )


def _deg_body(dst_ref, out_ref, idx_v, ones_v, zbuf, acc, ss0, ss1):
    c = lax.axis_index("c")
    s = lax.axis_index("s")
    wid = c * NS + s
    one16 = jnp.ones((16,), f32)
    zero16 = jnp.zeros((16,), f32)

    def fill_ones(i, _):
        ones_v[i, :] = one16
        return 0

    lax.fori_loop(0, K, fill_ones, 0)

    def fill_zero(i, _):
        zbuf[i, :] = zero16
        return 0

    lax.fori_loop(0, RPT, fill_zero, 0)
    pltpu.sync_copy(zbuf, acc.at[pl.ds(s * RPT, RPT)])
    pltpu.sync_copy(dst_ref.at[wid], idx_v)
    plsc.subcore_barrier()

    ssem = (ss0, ss1)

    def st(j, b):
        pltpu.async_copy(ones_v, acc.at[idx_v.at[j]], ssem[b], add=True)

    def ws(b):
        pltpu.make_async_copy(ones_v, acc.at[idx_v.at[0]], ssem[b]).wait()

    st(0, 0)

    def chunk(j2, _):
        j = 2 * j2 + 1
        st(j, 1)
        ws(0)
        st(j + 1, 0)
        ws(1)
        return 0

    lax.fori_loop(0, (NCH - 2) // 2, chunk, 0)
    st(NCH - 1, 1)
    ws(0)
    ws(1)
    plsc.subcore_barrier()
    pltpu.sync_copy(acc.at[pl.ds(s * RPT, RPT)], out_ref.at[c, pl.ds(s * RPT, RPT)])


def _deg_call(dst3):
    return pl.kernel(
        _deg_body,
        out_type=jax.ShapeDtypeStruct((NC, NPAD, 16), f32),
        mesh=_mesh(),
        compiler_params=pltpu.CompilerParams(use_tc_tiling_on_sc=False),
        scratch_types=[
            pltpu.VMEM((NCH, K), jnp.int32),
            pltpu.VMEM((K, 16), f32),
            pltpu.VMEM((RPT, 16), f32),
            pltpu.VMEM_SHARED((NPAD, 16), f32),
            pltpu.SemaphoreType.DMA,
            pltpu.SemaphoreType.DMA,
        ],
    )(dst3)


def _make_agg_body(D):
    def body(
        tbl_ref, src_ref, dst_ref, out_ref,
        src_v, dst_v, buf0, buf1, buf2, buf3, zbuf, acc,
        gs0, gs1, gs2, gs3, ss0, ss1, ss2, ss3,
    ):
        c = lax.axis_index("c")
        s = lax.axis_index("s")
        wid = c * NS + s
        bufs = (buf0, buf1, buf2, buf3)
        gsem = (gs0, gs1, gs2, gs3)
        ssem = (ss0, ss1, ss2, ss3)
        zero16 = jnp.zeros((16,), f32)

        def fill_zero(i, _):
            for l in range(D // 16):
                zbuf[i, pl.ds(l * 16, 16)] = zero16
            return 0

        lax.fori_loop(0, K, fill_zero, 0)
        for t in range(RPT // K):
            pltpu.sync_copy(zbuf, acc.at[pl.ds(s * RPT + t * K, K)])
        pltpu.sync_copy(src_ref.at[wid], src_v)
        pltpu.sync_copy(dst_ref.at[wid], dst_v)
        plsc.subcore_barrier()

        def sg(j, b):
            pltpu.async_copy(tbl_ref.at[src_v.at[j]], bufs[b], gsem[b])

        def wg(b):
            pltpu.make_async_copy(tbl_ref.at[src_v.at[0]], bufs[b], gsem[b]).wait()

        def st(j, b):
            pltpu.async_copy(bufs[b], acc.at[dst_v.at[j]], ssem[b], add=True)

        def ws(b):
            pltpu.make_async_copy(bufs[b], acc.at[dst_v.at[0]], ssem[b]).wait()

        sg(0, 0)
        sg(1, 1)
        wg(0)
        st(0, 0)
        sg(2, 2)
        wg(1)
        st(1, 1)
        sg(3, 3)

        def step(j, b):
            wg(b)
            st(j, b)
            bn = (b + 2) % 4
            ws(bn)
            sg(j + 2, bn)

        def main4(j4, _):
            base = 2 + 4 * j4
            step(base + 0, 2)
            step(base + 1, 3)
            step(base + 2, 0)
            step(base + 3, 1)
            return 0

        lax.fori_loop(0, (NCH - 4) // 4, main4, 0)
        wg(2)
        st(NCH - 2, 2)
        wg(3)
        st(NCH - 1, 3)
        ws(0)
        ws(1)
        ws(2)
        ws(3)
        plsc.subcore_barrier()
        pltpu.sync_copy(
            acc.at[pl.ds(s * RPT, RPT)], out_ref.at[c, pl.ds(s * RPT, RPT)]
        )

    return body


def _agg_call(table, src3, dst3, D):
    return pl.kernel(
        _make_agg_body(D),
        out_type=jax.ShapeDtypeStruct((NC, NPAD, D), f32),
        mesh=_mesh(),
        compiler_params=pltpu.CompilerParams(use_tc_tiling_on_sc=False),
        scratch_types=[
            pltpu.VMEM((NCH, K), jnp.int32),
            pltpu.VMEM((NCH, K), jnp.int32),
            pltpu.VMEM((K, D), f32),
            pltpu.VMEM((K, D), f32),
            pltpu.VMEM((K, D), f32),
            pltpu.VMEM((K, D), f32),
            pltpu.VMEM((K, D), f32),
            pltpu.VMEM_SHARED((NPAD, D), f32),
            pltpu.SemaphoreType.DMA,
            pltpu.SemaphoreType.DMA,
            pltpu.SemaphoreType.DMA,
            pltpu.SemaphoreType.DMA,
            pltpu.SemaphoreType.DMA,
            pltpu.SemaphoreType.DMA,
            pltpu.SemaphoreType.DMA,
            pltpu.SemaphoreType.DMA,
        ],
    )(table, src3, dst3)


def _tc1(x, W1, degparts):
    def body(x_ref, w_ref, dp_ref, h_ref, dinv_ref):
        dp = dp_ref[...]
        deg = dp[0, :N, 0:1] + dp[1, :N, 0:1] + 1.0
        dinv = lax.rsqrt(deg)
        h = jnp.dot(x_ref[...], w_ref[...], preferred_element_type=f32)
        h_ref[...] = h * dinv
        dinv_ref[...] = dinv

    return pl.pallas_call(
        body,
        out_shape=(
            jax.ShapeDtypeStruct((N, HID), f32),
            jax.ShapeDtypeStruct((N, 1), f32),
        ),
    )(x, W1, degparts)


def _tc2(h1s, agg, dinv, b1, W2p):
    def body(h_ref, a_ref, d_ref, b_ref, w_ref, o_ref):
        a = a_ref[...]
        dinv = d_ref[...]
        out1 = jnp.maximum(dinv * (a[0, :N] + a[1, :N] + h_ref[...]) + b_ref[...], 0.0)
        h2 = jnp.dot(out1, w_ref[...], preferred_element_type=f32)
        o_ref[...] = h2 * dinv

    return pl.pallas_call(body, out_shape=jax.ShapeDtypeStruct((N, NCP), f32))(
        h1s, agg, dinv, b1, W2p
    )


def _tc3(h2s, agg, dinv, b2):
    def body(h_ref, a_ref, d_ref, b_ref, o_ref):
        a = a_ref[...]
        logits = d_ref[...] * (a[0, :N] + a[1, :N] + h_ref[...])
        l = logits[:, :NCLASS] + b_ref[...]
        m = jnp.max(l, axis=-1, keepdims=True)
        e = jnp.exp(l - m)
        o_ref[...] = e / jnp.sum(e, axis=-1, keepdims=True)

    return pl.pallas_call(body, out_shape=jax.ShapeDtypeStruct((N, NCLASS), f32))(
        h2s, agg, dinv, b2
    )


@jax.jit
def kernel(x, edge_index, W1, b1, W2, b2):
    ppw = NCH * K - E // NW
    src_w = edge_index[0].reshape(NW, E // NW)
    dst_w = edge_index[1].reshape(NW, E // NW)
    pad_dst = jnp.broadcast_to(
        N + jnp.arange(ppw, dtype=jnp.int32) % (NPAD - N), (NW, ppw)
    )
    src3 = jnp.pad(src_w, ((0, 0), (0, ppw))).reshape(NW, NCH, K)
    dst3 = jnp.concatenate([dst_w, pad_dst], axis=1).reshape(NW, NCH, K)
    degparts = _deg_call(dst3)
    h1s, dinv = _tc1(x, W1, degparts)
    agg1 = _agg_call(h1s, src3, dst3, HID)
    W2p = jnp.pad(W2, ((0, 0), (0, NCP - NCLASS)))
    h2s = _tc2(h1s, agg1, dinv, b1.reshape(1, HID), W2p)
    agg2 = _agg_call(h2s, src3, dst3, NCP)
    return _tc3(h2s, agg2, dinv, b2.reshape(1, NCLASS))


# speedup vs baseline: 2.1574x; 2.0359x over previous
import functools

import jax
import jax.numpy as jnp
from jax import lax
from jax.experimental import pallas as pl
from jax.experimental.pallas import tpu as pltpu
from jax.experimental.pallas import tpu_sc as plsc

N = 10000
F_IN = 128
HID = 64
NCLASS = 40
NCP = 48
E = 320000
NC, NS = 2, 16
NW = NC * NS
K = 128
NCH = 80
EPAD = NW * NCH * K
NPAD = 10240
RPT = NPAD // NS

f32 = jnp.float32


def _mesh():
    return plsc.VectorSubcoreMesh(
        core_axis_name="c", subcore_axis_name="s", num_cores=NC, num_subcores=NS
    )


def _deg_body(dst_ref, out_ref, idx_v, ones_v, zbuf, acc, ss0, ss1):
    c = lax.axis_index("c")
    s = lax.axis_index("s")
    wid = c * NS + s
    one16 = jnp.ones((16,), f32)
    zero16 = jnp.zeros((16,), f32)

    def fill_ones(i, _):
        ones_v[i, :] = one16
        return 0

    lax.fori_loop(0, K, fill_ones, 0)

    def fill_zero(i, _):
        zbuf[i, :] = zero16
        return 0

    lax.fori_loop(0, RPT, fill_zero, 0)
    pltpu.sync_copy(zbuf, acc.at[pl.ds(s * RPT, RPT)])
    pltpu.sync_copy(dst_ref.at[wid], idx_v)
    plsc.subcore_barrier()

    ssem = (ss0, ss1)

    def st(j, b):
        pltpu.async_copy(ones_v, acc.at[idx_v.at[j]], ssem[b], add=True)

    def ws(b):
        pltpu.make_async_copy(ones_v, acc.at[idx_v.at[0]], ssem[b]).wait()

    st(0, 0)

    def chunk(j2, _):
        j = 2 * j2 + 1
        st(j, 1)
        ws(0)
        st(j + 1, 0)
        ws(1)
        return 0

    lax.fori_loop(0, (NCH - 2) // 2, chunk, 0)
    st(NCH - 1, 1)
    ws(0)
    ws(1)
    plsc.subcore_barrier()
    pltpu.sync_copy(acc.at[pl.ds(s * RPT, RPT)], out_ref.at[c, pl.ds(s * RPT, RPT)])


def _deg_call(dst3):
    return pl.kernel(
        _deg_body,
        out_type=jax.ShapeDtypeStruct((NC, NPAD, 16), f32),
        mesh=_mesh(),
        compiler_params=pltpu.CompilerParams(use_tc_tiling_on_sc=False),
        scratch_types=[
            pltpu.VMEM((NCH, K), jnp.int32),
            pltpu.VMEM((K, 16), f32),
            pltpu.VMEM((RPT, 16), f32),
            pltpu.VMEM_SHARED((NPAD, 16), f32),
            pltpu.SemaphoreType.DMA,
            pltpu.SemaphoreType.DMA,
        ],
    )(dst3)


def _make_agg_body(D):
    def body(
        tbl_ref, src_ref, dst_ref, out_ref,
        src_v, dst_v, buf0, buf1, buf2, buf3, zbuf, acc,
        gs0, gs1, gs2, gs3, ss0, ss1, ss2, ss3,
    ):
        c = lax.axis_index("c")
        s = lax.axis_index("s")
        wid = c * NS + s
        bufs = (buf0, buf1, buf2, buf3)
        gsem = (gs0, gs1, gs2, gs3)
        ssem = (ss0, ss1, ss2, ss3)
        zero16 = jnp.zeros((16,), f32)

        def fill_zero(i, _):
            for l in range(D // 16):
                zbuf[i, pl.ds(l * 16, 16)] = zero16
            return 0

        lax.fori_loop(0, K, fill_zero, 0)
        for t in range(RPT // K):
            pltpu.sync_copy(zbuf, acc.at[pl.ds(s * RPT + t * K, K)])
        pltpu.sync_copy(src_ref.at[wid], src_v)
        pltpu.sync_copy(dst_ref.at[wid], dst_v)
        plsc.subcore_barrier()

        def sg(j, b):
            pltpu.async_copy(tbl_ref.at[src_v.at[j]], bufs[b], gsem[b])

        def wg(b):
            pltpu.make_async_copy(tbl_ref.at[src_v.at[0]], bufs[b], gsem[b]).wait()

        def st(j, b):
            pltpu.async_copy(bufs[b], acc.at[dst_v.at[j]], ssem[b], add=True)

        def ws(b):
            pltpu.make_async_copy(bufs[b], acc.at[dst_v.at[0]], ssem[b]).wait()

        sg(0, 0)
        sg(1, 1)
        wg(0)
        st(0, 0)
        sg(2, 2)
        wg(1)
        st(1, 1)
        sg(3, 3)

        def step(j, b):
            wg(b)
            st(j, b)
            bn = (b + 2) % 4
            ws(bn)
            sg(j + 2, bn)

        def main4(j4, _):
            base = 2 + 4 * j4
            step(base + 0, 2)
            step(base + 1, 3)
            step(base + 2, 0)
            step(base + 3, 1)
            return 0

        lax.fori_loop(0, (NCH - 4) // 4, main4, 0)
        wg(2)
        st(NCH - 2, 2)
        wg(3)
        st(NCH - 1, 3)
        ws(0)
        ws(1)
        ws(2)
        ws(3)
        plsc.subcore_barrier()
        pltpu.sync_copy(
            acc.at[pl.ds(s * RPT, RPT)], out_ref.at[c, pl.ds(s * RPT, RPT)]
        )

    return body


def _agg_call(table, src3, dst3, D):
    return pl.kernel(
        _make_agg_body(D),
        out_type=jax.ShapeDtypeStruct((NC, NPAD, D), f32),
        mesh=_mesh(),
        compiler_params=pltpu.CompilerParams(use_tc_tiling_on_sc=False),
        scratch_types=[
            pltpu.VMEM((NCH, K), jnp.int32),
            pltpu.VMEM((NCH, K), jnp.int32),
            pltpu.VMEM((K, D), f32),
            pltpu.VMEM((K, D), f32),
            pltpu.VMEM((K, D), f32),
            pltpu.VMEM((K, D), f32),
            pltpu.VMEM((K, D), f32),
            pltpu.VMEM_SHARED((NPAD, D), f32),
            pltpu.SemaphoreType.DMA,
            pltpu.SemaphoreType.DMA,
            pltpu.SemaphoreType.DMA,
            pltpu.SemaphoreType.DMA,
            pltpu.SemaphoreType.DMA,
            pltpu.SemaphoreType.DMA,
            pltpu.SemaphoreType.DMA,
            pltpu.SemaphoreType.DMA,
        ],
    )(table, src3, dst3)


def _tc1(x, W1, degparts):
    def body(x_ref, w_ref, dp_ref, h_ref, dinv_ref):
        dp = dp_ref[...]
        deg = dp[0, :N, 0:1] + dp[1, :N, 0:1] + 1.0
        dinv = lax.rsqrt(deg)
        h = jnp.dot(x_ref[...], w_ref[...], preferred_element_type=f32)
        h_ref[...] = h * dinv
        dinv_ref[...] = dinv

    return pl.pallas_call(
        body,
        out_shape=(
            jax.ShapeDtypeStruct((N, HID), f32),
            jax.ShapeDtypeStruct((N, 1), f32),
        ),
    )(x, W1, degparts)


def _tc2(h1s, agg, dinv, b1, W2p):
    def body(h_ref, a_ref, d_ref, b_ref, w_ref, o_ref):
        a = a_ref[...]
        dinv = d_ref[...]
        out1 = jnp.maximum(dinv * (a[0, :N] + a[1, :N] + h_ref[...]) + b_ref[...], 0.0)
        h2 = jnp.dot(out1, w_ref[...], preferred_element_type=f32)
        o_ref[...] = h2 * dinv

    return pl.pallas_call(body, out_shape=jax.ShapeDtypeStruct((N, NCP), f32))(
        h1s, agg, dinv, b1, W2p
    )


def _tc3(h2s, agg, dinv, b2):
    def body(h_ref, a_ref, d_ref, b_ref, o_ref):
        a = a_ref[...]
        logits = d_ref[...] * (a[0, :N] + a[1, :N] + h_ref[...])
        l = logits[:, :NCLASS] + b_ref[...]
        m = jnp.max(l, axis=-1, keepdims=True)
        e = jnp.exp(l - m)
        o_ref[...] = e / jnp.sum(e, axis=-1, keepdims=True)

    return pl.pallas_call(body, out_shape=jax.ShapeDtypeStruct((N, NCLASS), f32))(
        h2s, agg, dinv, b2
    )


@jax.jit
def kernel(x, edge_index, W1, b1, W2, b2):
    ppw = NCH * K - E // NW
    src_w = edge_index[0].reshape(NW, E // NW)
    dst_w = edge_index[1].reshape(NW, E // NW)
    pad_dst = jnp.broadcast_to(
        N + jnp.arange(ppw, dtype=jnp.int32) % (NPAD - N), (NW, ppw)
    )
    pad_src = jnp.broadcast_to(
        jnp.arange(ppw, dtype=jnp.int32) * 37 % N, (NW, ppw)
    )
    src3 = jnp.concatenate([src_w, pad_src], axis=1).reshape(NW, NCH, K)
    dst3 = jnp.concatenate([dst_w, pad_dst], axis=1).reshape(NW, NCH, K)
    degparts = _deg_call(dst3)
    h1s, dinv = _tc1(x, W1, degparts)
    agg1 = _agg_call(h1s, src3, dst3, HID)
    W2p = jnp.pad(W2, ((0, 0), (0, NCP - NCLASS)))
    h2s = _tc2(h1s, agg1, dinv, b1.reshape(1, HID), W2p)
    agg2 = _agg_call(h2s, src3, dst3, NCP)
    return _tc3(h2s, agg2, dinv, b2.reshape(1, NCLASS))
